# baseline (speedup 1.0000x reference)
import functools

import jax
import jax.numpy as jnp
import numpy as np
from jax import lax
from jax.experimental import pallas as pl
from jax.experimental.pallas import tpu as pltpu
from jax.experimental.pallas import tpu_sc as plsc

POS_DIM = 3
N_LEVELS = 16
N_FEATS = 2
LOG2_HASHMAP_SIZE = 19
CAPACITY = 2 ** LOG2_HASHMAP_SIZE
COARSEST_RES = 16.0
FINEST_RES = 2048.0
N_POINTS = 262144
HASH_MUL = 2531011

C = 1024
NQ = 2
BN = C * NQ
G = N_POINTS // BN
G2 = N_POINTS // C
NW = 32
CPW = G2 // NW
RPL = CAPACITY // 8
SLICE = RPL // 16


def _scales():
    g = np.exp((np.log(FINEST_RES) - np.log(COARSEST_RES)) / (N_LEVELS - 1))
    level_scales = COARSEST_RES * g ** np.arange(N_LEVELS)
    inv_std = (POS_DIM + 1) * np.sqrt(2.0 / 3.0)
    base = np.array([1.0 / np.sqrt((i + 1.0) * (i + 2.0)) for i in range(POS_DIM)]) * inv_std
    return (level_scales[:, None] * base[None, :]).astype(np.float32)


SCALES = _scales()


def _stage1_body(x_ref, y_ref, z_ref, pk_ref):
    x = x_ref[0]
    y = y_ref[0]
    z = z_ref[0]
    for l in range(N_LEVELS):
        s0, s1, s2 = (float(SCALES[l, j]) for j in range(3))
        c0 = x * s0
        c1 = y * s1
        c2 = z * s2
        sm = c2 + c1
        e = [sm + c0, sm - c0, c2 - 2.0 * c1, -3.0 * c2]
        rem0f = []
        for k in range(4):
            v = e[k] * 0.25
            up = jnp.ceil(v) * 4.0
            dn = jnp.floor(v) * 4.0
            rem0f.append(jnp.where(up - e[k] < e[k] - dn, up, dn))
        rem0 = [jnp.round(r).astype(jnp.int32) for r in rem0f]
        _sum = jnp.round(
            (rem0f[0] + rem0f[1] + rem0f[2] + rem0f[3]) * 0.25).astype(jnp.int32)
        d = [e[k] - rem0f[k] for k in range(4)]
        lt01 = d[0] < d[1]
        lt02 = d[0] < d[2]
        lt03 = d[0] < d[3]
        lt12 = d[1] < d[2]
        lt13 = d[1] < d[3]
        lt23 = d[2] < d[3]
        bi = lambda m: m.astype(jnp.int32)
        rank = [bi(lt01) + bi(lt02) + bi(lt03),
                bi(~lt01) + bi(lt12) + bi(lt13),
                bi(~lt02) + bi(~lt12) + bi(lt23),
                bi(~lt03) + bi(~lt13) + bi(~lt23)]
        rank = [r + _sum for r in rank]
        for k in range(4):
            su = rank[k] < 0
            sd = rank[k] > 3
            shift = jnp.where(su, 4, jnp.where(sd, -4, 0))
            rank[k] = rank[k] + shift
            rem0[k] = rem0[k] + shift
        delta = [(e[k] - rem0[k].astype(jnp.float32)) * 0.25 for k in range(4)]

        def sel(rv):
            s = jnp.where(rank[0] == rv, delta[0], 0.0)
            for k in range(1, 4):
                s = s + jnp.where(rank[k] == rv, delta[k], 0.0)
            return s

        s3, s2_, s1_, s0_ = sel(3), sel(2), sel(1), sel(0)
        w = [1.0 + s3 - s0_, s2_ - s3, s1_ - s2_, s0_ - s1_]
        rem = None
        for r in range(4):
            h = jnp.zeros_like(rem0[0], dtype=jnp.uint32)
            for j in range(3):
                key = rem0[j] + r - jnp.where(rank[j] > 3 - r, 4, 0)
                h = (h + key.astype(jnp.uint32)) * jnp.uint32(HASH_MUL)
            hidx = (h & jnp.uint32(CAPACITY - 1)).astype(jnp.int32)
            pk_ref[0, :, 9 * l + r] = hidx >> 3
            pk_ref[0, :, 9 * l + 4 + r] = lax.bitcast_convert_type(w[r], jnp.int32)
            o = hidx & 7
            rem = o if r == 0 else rem | (o << (4 * r))
        pk_ref[0, :, 9 * l + 8] = rem


def _stage1(x, y, z):
    bs_in = pl.BlockSpec((1, NQ, 8, 128), lambda g: (g, 0, 0, 0))
    bs_out = pl.BlockSpec((1, NQ, 9 * N_LEVELS, 8, 128), lambda g: (g, 0, 0, 0, 0))
    return pl.pallas_call(
        _stage1_body,
        grid=(G,),
        in_specs=[bs_in, bs_in, bs_in],
        out_specs=[bs_out],
        out_shape=[
            jax.ShapeDtypeStruct((G, NQ, 9 * N_LEVELS, 8, 128), jnp.int32),
        ],
    )(x, y, z)


def _stage2_body(pk_hbm, tbl_hbm, o_hbm, *sc):
    pkb = [sc[0], sc[1]]
    rows = [[sc[2], sc[3], sc[4], sc[5]], [sc[6], sc[7], sc[8], sc[9]]]
    ovb = [[sc[10], sc[11]], [sc[12], sc[13]]]
    spb = [sc[14], sc[14]]
    sems_g = [sc[15], sc[16]]
    sems_st = [sc[17], sc[17]]
    sems_ov = [sc[18], sc[19]]
    sid = lax.axis_index("s")
    wid = sid * 2 + lax.axis_index("c")
    lanes = lax.iota(jnp.int32, 16)

    def stage_fire(l):
        sl = sid * SLICE
        pltpu.async_copy(tbl_hbm.at[l, pl.ds(sl, SLICE)],
                         spb[l % 2].at[pl.ds(sl, SLICE)], sems_st[l % 2])

    def stage_wait(l):
        sl = sid * SLICE
        pltpu.make_async_copy(tbl_hbm.at[l, pl.ds(sl, SLICE)],
                              spb[l % 2].at[pl.ds(sl, SLICE)],
                              sems_st[l % 2]).wait()

    def pk_copy(l, cch, p):
        pltpu.sync_copy(pk_hbm.at[wid * CPW + cch, l], pkb[p])

    def fire_g(l, p):
        for r in range(4):
            pltpu.async_copy(spb[l % 2].at[pkb[p].at[r]], rows[p][r],
                             sems_g[p])

    def drain_g(l, p):
        for r in range(4):
            pltpu.make_async_copy(spb[l % 2].at[pkb[p].at[r]], rows[p][r],
                                  sems_g[p]).wait()

    def out_fire(l, cch, p):
        base = (wid * CPW + cch) * C
        for f in range(2):
            pltpu.async_copy(ovb[p][f], o_hbm.at[f, l, pl.ds(base, C)],
                             sems_ov[p])

    def out_drain(l, cch, p):
        base = (wid * CPW + cch) * C
        for f in range(2):
            pltpu.make_async_copy(ovb[p][f], o_hbm.at[f, l, pl.ds(base, C)],
                                  sems_ov[p]).wait()

    def blend(p):
        def body(t, _):
            base = t * 16
            pidx = base + lanes
            rp = pkb[p][8, pl.ds(base, 16)]
            acc0 = jnp.zeros((16,), jnp.float32)
            acc1 = jnp.zeros((16,), jnp.float32)
            for r in range(4):
                wv = plsc.bitcast(pkb[p][4 + r, pl.ds(base, 16)], jnp.float32)
                off = (rp >> (4 * r)) & 7
                packed = plsc.load_gather(rows[p][r], [pidx, off])
                f0 = plsc.bitcast(packed << 16, jnp.float32)
                f1 = plsc.bitcast(packed & jnp.int32(-65536), jnp.float32)
                acc0 = acc0 + wv * f0
                acc1 = acc1 + wv * f1
            ovb[p][0][pl.ds(base, 16)] = acc0
            ovb[p][1][pl.ds(base, 16)] = acc1
            return 0
        lax.fori_loop(0, C // 16, body, 0)

    for l in range(N_LEVELS):
        if l > 0:
            plsc.subcore_barrier()
        stage_fire(l)
        stage_wait(l)
        plsc.subcore_barrier()
        pk_copy(l, 0, 0)
        fire_g(l, 0)

        def pair(m, _):
            cA = 2 * m
            drain_g(l, 0)
            pk_copy(l, cA + 1, 1)
            fire_g(l, 1)
            if l == 0:
                @pl.when(m >= 1)
                def _():
                    out_drain(l, cA, 0)
            else:
                out_drain(l, cA, 0)
            blend(0)
            out_fire(l, cA, 0)
            drain_g(l, 1)

            @pl.when(m < 3)
            def _():
                pk_copy(l, cA + 2, 0)
                fire_g(l, 0)
            if l == 0:
                @pl.when(m >= 1)
                def _():
                    out_drain(l, cA + 1, 1)
            else:
                out_drain(l, cA + 1, 1)
            blend(1)
            out_fire(l, cA + 1, 1)
            return 0

        lax.fori_loop(0, CPW // 2, pair, 0)

    out_drain(N_LEVELS - 1, CPW - 2, 0)
    out_drain(N_LEVELS - 1, CPW - 1, 1)


@functools.lru_cache(maxsize=None)
def _make_stage2():
    return pl.kernel(
        _stage2_body,
        out_type=jax.ShapeDtypeStruct((N_FEATS, N_LEVELS, N_POINTS), jnp.float32),
        mesh=plsc.VectorSubcoreMesh(core_axis_name="c", subcore_axis_name="s"),
        compiler_params=pltpu.CompilerParams(
            needs_layout_passes=False, use_tc_tiling_on_sc=False),
        scratch_types=(
            [pltpu.VMEM((9, C), jnp.int32)] * 2
            + [pltpu.VMEM((C, 8), jnp.int32)] * 8
            + [pltpu.VMEM((C,), jnp.float32)] * 4
            + [pltpu.VMEM_SHARED((RPL, 8), jnp.int32)] * 1
            + [pltpu.SemaphoreType.DMA] * 6
        ),
    )


def kernel(input, flattened_params):
    pos_t = input.T
    x = pos_t[0].reshape(G, NQ, 8, 128)
    y = pos_t[1].reshape(G, NQ, 8, 128)
    z = pos_t[2].reshape(G, NQ, 8, 128)
    (pk,) = _stage1(x, y, z)
    pk = pk.reshape(G2, N_LEVELS, 9, C)
    bf = flattened_params.astype(jnp.bfloat16).reshape(-1, 2)
    tbl = lax.bitcast_convert_type(bf, jnp.int32).reshape(N_LEVELS, RPL, 8)
    o = _make_stage2()(pk, tbl)
    return o.transpose(2, 1, 0).reshape(N_POINTS, N_LEVELS * N_FEATS)


# speedup vs baseline: 7.6796x; 7.6796x over previous
import functools

import jax
import jax.numpy as jnp
import numpy as np
from jax import lax
from jax.experimental import pallas as pl
from jax.experimental.pallas import tpu as pltpu
from jax.experimental.pallas import tpu_sc as plsc

POS_DIM = 3
N_LEVELS = 16
N_FEATS = 2
LOG2_HASHMAP_SIZE = 19
CAPACITY = 2 ** LOG2_HASHMAP_SIZE
COARSEST_RES = 16.0
FINEST_RES = 2048.0
N_POINTS = 262144
HASH_MUL = 2531011

C = 1024
NQ = 2
BN = C * NQ
G = N_POINTS // BN
G2 = N_POINTS // C
NW = 32
CPW = G2 // NW
ITERS = CPW * N_LEVELS


def _scales():
    g = np.exp((np.log(FINEST_RES) - np.log(COARSEST_RES)) / (N_LEVELS - 1))
    level_scales = COARSEST_RES * g ** np.arange(N_LEVELS)
    inv_std = (POS_DIM + 1) * np.sqrt(2.0 / 3.0)
    base = np.array([1.0 / np.sqrt((i + 1.0) * (i + 2.0)) for i in range(POS_DIM)]) * inv_std
    return (level_scales[:, None] * base[None, :]).astype(np.float32)


SCALES = _scales()


def _stage1_body(x_ref, y_ref, z_ref, pk_ref):
    x = x_ref[0]
    y = y_ref[0]
    z = z_ref[0]
    for l in range(N_LEVELS):
        s0, s1, s2 = (float(SCALES[l, j]) for j in range(3))
        c0 = x * s0
        c1 = y * s1
        c2 = z * s2
        sm = c2 + c1
        e = [sm + c0, sm - c0, c2 - 2.0 * c1, -3.0 * c2]
        rem0f = []
        for k in range(4):
            v = e[k] * 0.25
            up = jnp.ceil(v) * 4.0
            dn = jnp.floor(v) * 4.0
            rem0f.append(jnp.where(up - e[k] < e[k] - dn, up, dn))
        rem0 = [jnp.round(r).astype(jnp.int32) for r in rem0f]
        _sum = jnp.round(
            (rem0f[0] + rem0f[1] + rem0f[2] + rem0f[3]) * 0.25).astype(jnp.int32)
        d = [e[k] - rem0f[k] for k in range(4)]
        lt01 = d[0] < d[1]
        lt02 = d[0] < d[2]
        lt03 = d[0] < d[3]
        lt12 = d[1] < d[2]
        lt13 = d[1] < d[3]
        lt23 = d[2] < d[3]
        bi = lambda m: m.astype(jnp.int32)
        rank = [bi(lt01) + bi(lt02) + bi(lt03),
                bi(~lt01) + bi(lt12) + bi(lt13),
                bi(~lt02) + bi(~lt12) + bi(lt23),
                bi(~lt03) + bi(~lt13) + bi(~lt23)]
        rank = [r + _sum for r in rank]
        for k in range(4):
            su = rank[k] < 0
            sd = rank[k] > 3
            shift = jnp.where(su, 4, jnp.where(sd, -4, 0))
            rank[k] = rank[k] + shift
            rem0[k] = rem0[k] + shift
        delta = [(e[k] - rem0[k].astype(jnp.float32)) * 0.25 for k in range(4)]

        def sel(rv):
            s = jnp.where(rank[0] == rv, delta[0], 0.0)
            for k in range(1, 4):
                s = s + jnp.where(rank[k] == rv, delta[k], 0.0)
            return s

        s3, s2_, s1_, s0_ = sel(3), sel(2), sel(1), sel(0)
        w = [1.0 + s3 - s0_, s2_ - s3, s1_ - s2_, s0_ - s1_]
        rem = None
        for r in range(4):
            h = jnp.zeros_like(rem0[0], dtype=jnp.uint32)
            for j in range(3):
                key = rem0[j] + r - jnp.where(rank[j] > 3 - r, 4, 0)
                h = (h + key.astype(jnp.uint32)) * jnp.uint32(HASH_MUL)
            hidx = (h & jnp.uint32(CAPACITY - 1)).astype(jnp.int32)
            row = hidx + l * CAPACITY
            pk_ref[0, :, 9 * l + r] = row >> 2
            pk_ref[0, :, 9 * l + 4 + r] = lax.bitcast_convert_type(w[r], jnp.int32)
            o = (row & 3) * 2
            rem = o if r == 0 else rem | (o << (4 * r))
        pk_ref[0, :, 9 * l + 8] = rem


def _stage1(x, y, z):
    g_count = x.shape[0]
    bs_in = pl.BlockSpec((1, NQ, 8, 128), lambda g: (g, 0, 0, 0))
    bs_out = pl.BlockSpec((1, NQ, 9 * N_LEVELS, 8, 128), lambda g: (g, 0, 0, 0, 0))
    return pl.pallas_call(
        _stage1_body,
        grid=(g_count,),
        in_specs=[bs_in, bs_in, bs_in],
        out_specs=[bs_out],
        out_shape=[
            jax.ShapeDtypeStruct((g_count, NQ, 9 * N_LEVELS, 8, 128), jnp.int32),
        ],
    )(x, y, z)


def _stage2_body(cpw, pk_hbm, table_hbm, out_hbm,
                 pk0, pk1, *rest):
    rows = [[rest[0], rest[1], rest[2], rest[3]],
            [rest[4], rest[5], rest[6], rest[7]]]
    out_g, sem0, sem1 = rest[8], rest[9], rest[10]
    pkb = [pk0, pk1]
    sems = [sem0, sem1]
    wid = lax.axis_index("s") * 2 + lax.axis_index("c")
    lanes = lax.iota(jnp.int32, 16)

    def pk_src(it):
        return pk_hbm.at[wid * cpw + it // N_LEVELS, it % N_LEVELS]

    def fire(it, p):
        return [pltpu.async_copy(table_hbm.at[pkb[p].at[r]], rows[p][r], sems[p])
                for r in range(4)]

    pltpu.sync_copy(pk_src(0), pkb[0])
    fire(0, 0)

    def sub_iter(it, p):
        @pl.when(it < cpw * N_LEVELS - 1)
        def _():
            pltpu.sync_copy(pk_src(it + 1), pkb[1 - p])
            fire(it + 1, 1 - p)
        for r in range(4):
            pltpu.make_async_copy(
                table_hbm.at[pkb[p].at[r]], rows[p][r], sems[p]).wait()
        l = it % N_LEVELS
        col0 = jnp.full((16,), 2 * l, jnp.int32)
        col1 = col0 + 1

        def blend(t, _):
            base = t * 16
            pidx = base + lanes
            rp = pkb[p][8, pl.ds(base, 16)]
            acc0 = jnp.zeros((16,), jnp.float32)
            acc1 = jnp.zeros((16,), jnp.float32)
            for r in range(4):
                wv = plsc.bitcast(pkb[p][4 + r, pl.ds(base, 16)], jnp.float32)
                off = (rp >> (4 * r)) & 7
                f0 = plsc.load_gather(rows[p][r], [pidx, off])
                f1 = plsc.load_gather(rows[p][r], [pidx, off + 1])
                acc0 = acc0 + wv * f0
                acc1 = acc1 + wv * f1
            plsc.store_scatter(out_g, [pidx, col0], acc0)
            plsc.store_scatter(out_g, [pidx, col1], acc1)
            return 0

        lax.fori_loop(0, C // 16, blend, 0)

        @pl.when(l == N_LEVELS - 1)
        def _():
            base_pt = (wid * cpw + it // N_LEVELS) * C
            pltpu.sync_copy(out_g, out_hbm.at[pl.ds(base_pt, C), :])

    def macro(m, _):
        sub_iter(2 * m, 0)
        sub_iter(2 * m + 1, 1)
        return 0

    lax.fori_loop(0, cpw * N_LEVELS // 2, macro, 0)


@functools.lru_cache(maxsize=None)
def _make_stage2(npts, cpw):
    return pl.kernel(
        functools.partial(_stage2_body, cpw),
        out_type=jax.ShapeDtypeStruct((npts, N_LEVELS * N_FEATS), jnp.float32),
        mesh=plsc.VectorSubcoreMesh(core_axis_name="c", subcore_axis_name="s"),
        compiler_params=pltpu.CompilerParams(
            needs_layout_passes=False, use_tc_tiling_on_sc=False),
        scratch_types=(
            [pltpu.VMEM((9, C), jnp.int32)] * 2
            + [pltpu.VMEM((C, 8), jnp.float32)] * 8
            + [pltpu.VMEM((C, N_LEVELS * N_FEATS), jnp.float32)]
            + [pltpu.SemaphoreType.DMA] * 2
        ),
    )


def kernel(input, flattened_params):
    pos_t = input.T
    x = pos_t[0].reshape(G, NQ, 8, 128)
    y = pos_t[1].reshape(G, NQ, 8, 128)
    z = pos_t[2].reshape(G, NQ, 8, 128)
    table = flattened_params.reshape(N_LEVELS * CAPACITY * N_FEATS // 8, 8)
    halves = []
    for h in range(2):
        sl = slice(h * G // 2, (h + 1) * G // 2)
        (pk,) = _stage1(x[sl], y[sl], z[sl])
        pk = pk.reshape(G2 // 2, N_LEVELS, 9, C)
        halves.append(_make_stage2(N_POINTS // 2, CPW // 2)(pk, table))
    return jnp.concatenate(halves, axis=0)


# speedup vs baseline: 7.6935x; 1.0018x over previous
import functools

import jax
import jax.numpy as jnp
import numpy as np
from jax import lax
from jax.experimental import pallas as pl
from jax.experimental.pallas import tpu as pltpu
from jax.experimental.pallas import tpu_sc as plsc

POS_DIM = 3
N_LEVELS = 16
N_FEATS = 2
LOG2_HASHMAP_SIZE = 19
CAPACITY = 2 ** LOG2_HASHMAP_SIZE
COARSEST_RES = 16.0
FINEST_RES = 2048.0
N_POINTS = 262144
HASH_MUL = 2531011

C = 1024
NQ = 2
BN = C * NQ
G = N_POINTS // BN
G2 = N_POINTS // C
NW = 32
CPW = G2 // NW
ITERS = CPW * N_LEVELS


def _scales():
    g = np.exp((np.log(FINEST_RES) - np.log(COARSEST_RES)) / (N_LEVELS - 1))
    level_scales = COARSEST_RES * g ** np.arange(N_LEVELS)
    inv_std = (POS_DIM + 1) * np.sqrt(2.0 / 3.0)
    base = np.array([1.0 / np.sqrt((i + 1.0) * (i + 2.0)) for i in range(POS_DIM)]) * inv_std
    return (level_scales[:, None] * base[None, :]).astype(np.float32)


SCALES = _scales()


def _stage1_body(x_ref, y_ref, z_ref, pk_ref):
    x = x_ref[0]
    y = y_ref[0]
    z = z_ref[0]
    for l in range(N_LEVELS):
        s0, s1, s2 = (float(SCALES[l, j]) for j in range(3))
        c0 = x * s0
        c1 = y * s1
        c2 = z * s2
        sm = c2 + c1
        e = [sm + c0, sm - c0, c2 - 2.0 * c1, -3.0 * c2]
        rem0f = []
        for k in range(4):
            v = e[k] * 0.25
            up = jnp.ceil(v) * 4.0
            dn = jnp.floor(v) * 4.0
            rem0f.append(jnp.where(up - e[k] < e[k] - dn, up, dn))
        rem0 = [jnp.round(r).astype(jnp.int32) for r in rem0f]
        _sum = jnp.round(
            (rem0f[0] + rem0f[1] + rem0f[2] + rem0f[3]) * 0.25).astype(jnp.int32)
        d = [e[k] - rem0f[k] for k in range(4)]
        lt01 = d[0] < d[1]
        lt02 = d[0] < d[2]
        lt03 = d[0] < d[3]
        lt12 = d[1] < d[2]
        lt13 = d[1] < d[3]
        lt23 = d[2] < d[3]
        bi = lambda m: m.astype(jnp.int32)
        rank = [bi(lt01) + bi(lt02) + bi(lt03),
                bi(~lt01) + bi(lt12) + bi(lt13),
                bi(~lt02) + bi(~lt12) + bi(lt23),
                bi(~lt03) + bi(~lt13) + bi(~lt23)]
        rank = [r + _sum for r in rank]
        for k in range(4):
            su = rank[k] < 0
            sd = rank[k] > 3
            shift = jnp.where(su, 4, jnp.where(sd, -4, 0))
            rank[k] = rank[k] + shift
            rem0[k] = rem0[k] + shift
        delta = [(e[k] - rem0[k].astype(jnp.float32)) * 0.25 for k in range(4)]

        def sel(rv):
            s = jnp.where(rank[0] == rv, delta[0], 0.0)
            for k in range(1, 4):
                s = s + jnp.where(rank[k] == rv, delta[k], 0.0)
            return s

        s3, s2_, s1_, s0_ = sel(3), sel(2), sel(1), sel(0)
        w = [1.0 + s3 - s0_, s2_ - s3, s1_ - s2_, s0_ - s1_]
        rem = None
        for r in range(4):
            h = jnp.zeros_like(rem0[0], dtype=jnp.uint32)
            for j in range(3):
                key = rem0[j] + r - jnp.where(rank[j] > 3 - r, 4, 0)
                h = (h + key.astype(jnp.uint32)) * jnp.uint32(HASH_MUL)
            hidx = (h & jnp.uint32(CAPACITY - 1)).astype(jnp.int32)
            row = hidx + l * CAPACITY
            pk_ref[0, :, 9 * l + r] = row >> 2
            pk_ref[0, :, 9 * l + 4 + r] = lax.bitcast_convert_type(w[r], jnp.int32)
            o = (row & 3) * 2
            rem = o if r == 0 else rem | (o << (4 * r))
        pk_ref[0, :, 9 * l + 8] = rem


def _stage1(x, y, z):
    g_count = x.shape[0]
    bs_in = pl.BlockSpec((1, NQ, 8, 128), lambda g: (g, 0, 0, 0))
    bs_out = pl.BlockSpec((1, NQ, 9 * N_LEVELS, 8, 128), lambda g: (g, 0, 0, 0, 0))
    return pl.pallas_call(
        _stage1_body,
        grid=(g_count,),
        in_specs=[bs_in, bs_in, bs_in],
        out_specs=[bs_out],
        out_shape=[
            jax.ShapeDtypeStruct((g_count, NQ, 9 * N_LEVELS, 8, 128), jnp.int32),
        ],
    )(x, y, z)


def _stage2_body(cpw, pk_hbm, table_hbm, out_hbm,
                 pk0, pk1, *rest):
    rows = [[rest[0], rest[1], rest[2], rest[3]],
            [rest[4], rest[5], rest[6], rest[7]]]
    out_g, sem0, sem1 = rest[8], rest[9], rest[10]
    pkb = [pk0, pk1]
    sems = [sem0, sem1]
    wid = lax.axis_index("s") * 2 + lax.axis_index("c")
    lanes = lax.iota(jnp.int32, 16)

    def pk_src(it):
        return pk_hbm.at[wid * cpw + it // N_LEVELS, it % N_LEVELS]

    def fire(it, p):
        return [pltpu.async_copy(table_hbm.at[pkb[p].at[r]], rows[p][r], sems[p])
                for r in range(4)]

    pltpu.sync_copy(pk_src(0), pkb[0])
    fire(0, 0)

    def sub_iter(it, p):
        @pl.when(it < cpw * N_LEVELS - 1)
        def _():
            pltpu.sync_copy(pk_src(it + 1), pkb[1 - p])
            fire(it + 1, 1 - p)
        for r in range(4):
            pltpu.make_async_copy(
                table_hbm.at[pkb[p].at[r]], rows[p][r], sems[p]).wait()
        l = it % N_LEVELS
        col0 = jnp.full((16,), 2 * l, jnp.int32)
        col1 = col0 + 1

        def blend(t, _):
            base = t * 16
            pidx = base + lanes
            rp = pkb[p][8, pl.ds(base, 16)]
            acc0 = jnp.zeros((16,), jnp.float32)
            acc1 = jnp.zeros((16,), jnp.float32)
            for r in range(4):
                wv = plsc.bitcast(pkb[p][4 + r, pl.ds(base, 16)], jnp.float32)
                off = (rp >> (4 * r)) & 7
                f0 = plsc.load_gather(rows[p][r], [pidx, off])
                f1 = plsc.load_gather(rows[p][r], [pidx, off + 1])
                acc0 = acc0 + wv * f0
                acc1 = acc1 + wv * f1
            plsc.store_scatter(out_g, [pidx, col0], acc0)
            plsc.store_scatter(out_g, [pidx, col1], acc1)
            return 0

        lax.fori_loop(0, C // 16, blend, 0)

        @pl.when(l == N_LEVELS - 1)
        def _():
            base_pt = (wid * cpw + it // N_LEVELS) * C
            pltpu.sync_copy(out_g, out_hbm.at[pl.ds(base_pt, C), :])

    def macro(m, _):
        sub_iter(2 * m, 0)
        sub_iter(2 * m + 1, 1)
        return 0

    lax.fori_loop(0, cpw * N_LEVELS // 2, macro, 0)


@functools.lru_cache(maxsize=None)
def _make_stage2(npts, cpw):
    return pl.kernel(
        functools.partial(_stage2_body, cpw),
        out_type=jax.ShapeDtypeStruct((npts, N_LEVELS * N_FEATS), jnp.float32),
        mesh=plsc.VectorSubcoreMesh(core_axis_name="c", subcore_axis_name="s"),
        compiler_params=pltpu.CompilerParams(
            needs_layout_passes=False, use_tc_tiling_on_sc=False),
        scratch_types=(
            [pltpu.VMEM((9, C), jnp.int32)] * 2
            + [pltpu.VMEM((C, 8), jnp.float32)] * 8
            + [pltpu.VMEM((C, N_LEVELS * N_FEATS), jnp.float32)]
            + [pltpu.SemaphoreType.DMA] * 2
        ),
    )


def kernel(input, flattened_params):
    pos_t = input.T
    x = pos_t[0].reshape(G, NQ, 8, 128)
    y = pos_t[1].reshape(G, NQ, 8, 128)
    z = pos_t[2].reshape(G, NQ, 8, 128)
    table = flattened_params.reshape(N_LEVELS * CAPACITY * N_FEATS // 8, 8)
    halves = []
    for h in range(4):
        sl = slice(h * G // 4, (h + 1) * G // 4)
        (pk,) = _stage1(x[sl], y[sl], z[sl])
        pk = pk.reshape(G2 // 4, N_LEVELS, 9, C)
        halves.append(_make_stage2(N_POINTS // 4, CPW // 4)(pk, table))
    return jnp.concatenate(halves, axis=0)
